# baseline (speedup 1.0000x reference)
import functools

import jax
import jax.numpy as jnp
from jax import lax
from jax.experimental import pallas as pl
from jax.experimental.pallas import tpu as pltpu
from jax.experimental.pallas import tpu_sc as plsc

_E = 320000
_NG = 50000
_NM = 10000
_D = 128

_B = 1280
_W = 128
_EH = _E // 2
_NB = _EH // _B

_GBLK = 1000
_C = 128
_KBUF = 3
_NW = 32
_NMT = 624
_NMR = _NM - 16 * _NMT


def _ln(y, g, b):
    m = jnp.mean(y, axis=-1, keepdims=True)
    c = y - m
    v = jnp.mean(c * c, axis=-1, keepdims=True)
    return c * lax.rsqrt(v + 1e-5) * g + b


def _proj_body(x_ref, w_ref, out_ref):
    out_ref[...] = jnp.dot(x_ref[...], w_ref[...],
                           preferred_element_type=jnp.float32)


def _grid_proj_kernel(grid_nfeat, Ws):
    n = grid_nfeat.shape[0]
    return pl.pallas_call(
        _proj_body,
        grid=(n // _GBLK,),
        in_specs=[pl.BlockSpec((_GBLK, _D), lambda i: (i, 0)),
                  pl.BlockSpec((_D, _D), lambda i: (0, 0))],
        out_specs=pl.BlockSpec((_GBLK, _D), lambda i: (i, 0)),
        out_shape=jax.ShapeDtypeStruct((n, _D), jnp.float32),
        compiler_params=pltpu.CompilerParams(
            dimension_semantics=("parallel",)),
    )(grid_nfeat, Ws)


def _mesh_pre_body(x_ref, wd_ref, be1_ref, out_ref):
    out_ref[...] = (jnp.dot(x_ref[...], wd_ref[...],
                            preferred_element_type=jnp.float32) + be1_ref[...])


def _mesh_pre(mesh_nfeat, Wd, be1):
    n = mesh_nfeat.shape[0]
    return pl.pallas_call(
        _mesh_pre_body,
        grid=(n // _GBLK,),
        in_specs=[pl.BlockSpec((_GBLK, _D), lambda i: (i, 0)),
                  pl.BlockSpec((_D, _D), lambda i: (0, 0)),
                  pl.BlockSpec((1, _D), lambda i: (0, 0))],
        out_specs=pl.BlockSpec((_GBLK, _D), lambda i: (i, 0)),
        out_shape=jax.ShapeDtypeStruct((n, _D), jnp.float32),
        compiler_params=pltpu.CompilerParams(
            dimension_semantics=("parallel",)),
    )(mesh_nfeat, Wd, be1.reshape(1, _D))


def _grid_body(x_ref, ws1_ref, bs1_ref, ws2_ref, bs2_ref, gs_ref, bgs_ref,
               out_ref):
    x = x_ref[...]
    hs = jax.nn.silu(
        jnp.dot(x, ws1_ref[...], preferred_element_type=jnp.float32)
        + bs1_ref[...])
    y = jnp.dot(hs, ws2_ref[...], preferred_element_type=jnp.float32) + bs2_ref[...]
    out_ref[...] = x + _ln(y, gs_ref[...], bgs_ref[...])


def _grid_branch(grid_nfeat, Ws1, bs1, Ws2, bs2, gs, bgs):
    n = grid_nfeat.shape[0]
    full = lambda i: (0, 0)
    wspec = pl.BlockSpec((_D, _D), full)
    vspec = pl.BlockSpec((1, _D), full)
    return pl.pallas_call(
        _grid_body,
        grid=(n // _GBLK,),
        in_specs=[pl.BlockSpec((_GBLK, _D), lambda i: (i, 0)), wspec, vspec,
                  wspec, vspec, vspec, vspec],
        out_specs=pl.BlockSpec((_GBLK, _D), lambda i: (i, 0)),
        out_shape=jax.ShapeDtypeStruct((n, _D), jnp.float32),
        compiler_params=pltpu.CompilerParams(
            dimension_semantics=("parallel",)),
    )(grid_nfeat, Ws1, bs1.reshape(1, _D), Ws2, bs2.reshape(1, _D),
      gs.reshape(1, _D), bgs.reshape(1, _D))


def _gather_loop(per_w, n_groups, tail, table, idx_hbm, out_hbm,
                 idx_v, rows_v, tail_idx, tail_rows, sem_g, sem_o, wid):
    base = wid * per_w

    def group(g, carry):
        @pl.when(g > 0)
        def _():
            for b in range(_KBUF):
                off = base + ((g - 1) * _KBUF + b) * _C
                pltpu.make_async_copy(
                    rows_v.at[b], out_hbm.at[pl.ds(off, _C)], sem_o).wait()

        for b in range(_KBUF):
            off = base + (g * _KBUF + b) * _C
            pltpu.sync_copy(idx_hbm.at[pl.ds(off, _C)], idx_v.at[b])
        for b in range(_KBUF):
            pltpu.make_async_copy(
                table.at[idx_v.at[b]], rows_v.at[b], sem_g).start()
        for b in range(_KBUF):
            pltpu.make_async_copy(
                table.at[idx_v.at[b]], rows_v.at[b], sem_g).wait()
        for b in range(_KBUF):
            off = base + (g * _KBUF + b) * _C
            pltpu.make_async_copy(
                rows_v.at[b], out_hbm.at[pl.ds(off, _C)], sem_o).start()
        return carry

    lax.fori_loop(0, n_groups, group, 0)
    for b in range(_KBUF):
        off = base + ((n_groups - 1) * _KBUF + b) * _C
        pltpu.make_async_copy(
            rows_v.at[b], out_hbm.at[pl.ds(off, _C)], sem_o).wait()
    if tail:
        off = base + n_groups * _KBUF * _C
        pltpu.sync_copy(idx_hbm.at[pl.ds(off, tail)], tail_idx)
        pltpu.async_copy(table.at[tail_idx], tail_rows, sem_g).wait()
        pltpu.sync_copy(tail_rows, out_hbm.at[pl.ds(off, tail)])


def _gather_split(n):
    per_w = n // _NW
    n_groups = per_w // (_C * _KBUF)
    tail = per_w - n_groups * _KBUF * _C
    return per_w, n_groups, tail


def _sc_gather_body(per_w, n_groups, tail, table_hbm, idx_hbm, out_hbm,
                    idx_v, rows_v, tail_idx, tail_rows, sem_g, sem_o):
    wid = lax.axis_index("s") * 2 + lax.axis_index("c")
    _gather_loop(per_w, n_groups, tail, table_hbm, idx_hbm, out_hbm,
                 idx_v, rows_v, tail_idx, tail_rows, sem_g, sem_o, wid)


def _sc_gather(table, idx):
    n = idx.shape[0]
    d = table.shape[1]
    per_w, n_groups, tail = _gather_split(n)
    mesh = plsc.VectorSubcoreMesh(core_axis_name="c", subcore_axis_name="s")
    kern = functools.partial(
        pl.kernel,
        mesh=mesh,
        out_type=jax.ShapeDtypeStruct((n, d), table.dtype),
        scratch_types=[
            pltpu.VMEM((_KBUF, _C), jnp.int32),
            pltpu.VMEM((_KBUF, _C, d), table.dtype),
            pltpu.VMEM((max(tail, 8),), jnp.int32),
            pltpu.VMEM((max(tail, 8), d), table.dtype),
            pltpu.SemaphoreType.DMA,
            pltpu.SemaphoreType.DMA,
        ],
    )(functools.partial(_sc_gather_body, per_w, n_groups, tail))
    return kern(table, idx)


def _sc_gather_sp_body(per_w, n_groups, tail, table_hbm, idx_hbm,
                       out_hbm, idx_v, rows_v, tail_idx, tail_rows, table_sh,
                       sem_g, sem_o):
    sid = lax.axis_index("s")
    wid = sid * 2 + lax.axis_index("c")

    pltpu.sync_copy(table_hbm.at[pl.ds(sid * _NMT, _NMT)],
                    table_sh.at[pl.ds(sid * _NMT, _NMT)])

    @pl.when(sid == 15)
    def _():
        pltpu.sync_copy(table_hbm.at[pl.ds(16 * _NMT, _NMR)],
                        table_sh.at[pl.ds(16 * _NMT, _NMR)])

    plsc.subcore_barrier()
    _gather_loop(per_w, n_groups, tail, table_sh, idx_hbm, out_hbm,
                 idx_v, rows_v, tail_idx, tail_rows, sem_g, sem_o, wid)


def _sc_gather_spmem(table, idx):
    n = idx.shape[0]
    d = table.shape[1]
    per_w, n_groups, tail = _gather_split(n)
    mesh = plsc.VectorSubcoreMesh(core_axis_name="c", subcore_axis_name="s")
    kern = functools.partial(
        pl.kernel,
        mesh=mesh,
        out_type=jax.ShapeDtypeStruct((n, d), table.dtype),
        scratch_types=[
            pltpu.VMEM((_KBUF, _C), jnp.int32),
            pltpu.VMEM((_KBUF, _C, d), table.dtype),
            pltpu.VMEM((max(tail, 8),), jnp.int32),
            pltpu.VMEM((max(tail, 8), d), table.dtype),
            pltpu.VMEM_SHARED((table.shape[0], d), table.dtype),
            pltpu.SemaphoreType.DMA,
            pltpu.SemaphoreType.DMA,
        ],
    )(functools.partial(_sc_gather_sp_body, per_w, n_groups, tail))
    return kern(table, idx)


def _sc_scatter_body(per_w, n_groups, tail, mlp_hbm, dst_hbm, zero_hbm,
                     out_hbm, idx_v, rows_v, tail_idx, tail_rows, acc_sh,
                     sem_g):
    cid = lax.axis_index("c")
    sid = lax.axis_index("s")
    wid = cid * 16 + sid
    base = wid * per_w

    pltpu.sync_copy(zero_hbm.at[pl.ds(sid * _NMT, _NMT)],
                    acc_sh.at[pl.ds(sid * _NMT, _NMT)])

    @pl.when(sid == 15)
    def _():
        pltpu.sync_copy(zero_hbm.at[pl.ds(16 * _NMT, _NMR)],
                        acc_sh.at[pl.ds(16 * _NMT, _NMR)])

    plsc.subcore_barrier()

    def group(g, carry):
        for b in range(_KBUF):
            off = base + (g * _KBUF + b) * _C
            pltpu.sync_copy(dst_hbm.at[pl.ds(off, _C)], idx_v.at[b])
        for b in range(_KBUF):
            off = base + (g * _KBUF + b) * _C
            pltpu.make_async_copy(
                mlp_hbm.at[pl.ds(off, _C)], rows_v.at[b], sem_g).start()
        for b in range(_KBUF):
            off = base + (g * _KBUF + b) * _C
            pltpu.make_async_copy(
                mlp_hbm.at[pl.ds(off, _C)], rows_v.at[b], sem_g).wait()
            pltpu.sync_copy(rows_v.at[b], acc_sh.at[idx_v.at[b]], add=True)
        return carry

    lax.fori_loop(0, n_groups, group, 0)
    if tail:
        off = base + n_groups * _KBUF * _C
        pltpu.sync_copy(dst_hbm.at[pl.ds(off, tail)], tail_idx)
        pltpu.sync_copy(mlp_hbm.at[pl.ds(off, tail)], tail_rows)
        pltpu.sync_copy(tail_rows, acc_sh.at[tail_idx], add=True)
    plsc.subcore_barrier()
    pltpu.sync_copy(acc_sh.at[pl.ds(sid * _NMT, _NMT)],
                    out_hbm.at[cid, pl.ds(sid * _NMT, _NMT)])

    @pl.when(sid == 15)
    def _():
        pltpu.sync_copy(acc_sh.at[pl.ds(16 * _NMT, _NMR)],
                        out_hbm.at[cid, pl.ds(16 * _NMT, _NMR)])


def _sc_scatter(mlp, dst_idx_half, zeros_nm):
    n = mlp.shape[0]
    per_w = n // _NW
    n_groups = per_w // (_C * _KBUF)
    tail = per_w - n_groups * _KBUF * _C
    mesh = plsc.VectorSubcoreMesh(core_axis_name="c", subcore_axis_name="s")
    kern = functools.partial(
        pl.kernel,
        mesh=mesh,
        out_type=jax.ShapeDtypeStruct((2, _NM, _D), jnp.float32),
        scratch_types=[
            pltpu.VMEM((_KBUF, _C), jnp.int32),
            pltpu.VMEM((_KBUF, _C, _D), jnp.float32),
            pltpu.VMEM((max(tail, 8),), jnp.int32),
            pltpu.VMEM((max(tail, 8), _D), jnp.float32),
            pltpu.VMEM_SHARED((_NM, _D), jnp.float32),
            pltpu.SemaphoreType.DMA,
        ],
    )(functools.partial(_sc_scatter_body, per_w, n_groups, tail))
    return kern(mlp, dst_idx_half, zeros_nm)


def _edge_body(e_ref, gsrc_ref, gdst_ref, we_ref, we2_ref, be2_ref, ge_ref,
               bge_ref, out_ref):
    we_bf = we_ref[...].astype(jnp.bfloat16)
    we2_bf = we2_ref[...].astype(jnp.bfloat16)
    e_bf = e_ref[...].astype(jnp.bfloat16)
    h = jax.nn.silu(
        jnp.dot(e_bf, we_bf, preferred_element_type=jnp.float32)
        + gsrc_ref[...] + gdst_ref[...])
    y = (jnp.dot(h.astype(jnp.bfloat16), we2_bf,
                 preferred_element_type=jnp.float32) + be2_ref[...])
    out_ref[...] = _ln(y, ge_ref[...], bge_ref[...])


def _edge_mlp(e_half, gsrc, gdst, We, We2, be2, ge, bge):
    row = lambda i: (i, 0)
    full = lambda i: (0, 0)
    return pl.pallas_call(
        _edge_body,
        grid=(_NB,),
        in_specs=[
            pl.BlockSpec((_B, _D), row),
            pl.BlockSpec((_B, _D), row),
            pl.BlockSpec((_B, _D), row),
            pl.BlockSpec((_D, _D), full),
            pl.BlockSpec((_D, _D), full),
            pl.BlockSpec((1, _D), full),
            pl.BlockSpec((1, _D), full),
            pl.BlockSpec((1, _D), full),
        ],
        out_specs=pl.BlockSpec((_B, _D), row),
        out_shape=jax.ShapeDtypeStruct((e_half.shape[0], _D), jnp.float32),
        compiler_params=pltpu.CompilerParams(
            dimension_semantics=("parallel",)),
    )(e_half, gsrc, gdst, We, We2, be2.reshape(1, _D), ge.reshape(1, _D),
      bge.reshape(1, _D))


def _mesh_post_body(agg0_ref, agg1_ref, x_ref, wd1a_ref, wd1b_ref, bd1_ref,
                    wd2_ref, bd2_ref, gd_ref, bgd_ref, out_ref):
    x = x_ref[...]
    agg = (agg0_ref[0] + agg0_ref[1]) + (agg1_ref[0] + agg1_ref[1])
    hd = jax.nn.silu(
        jnp.dot(agg, wd1a_ref[...], preferred_element_type=jnp.float32)
        + jnp.dot(x, wd1b_ref[...], preferred_element_type=jnp.float32)
        + bd1_ref[...])
    y = jnp.dot(hd, wd2_ref[...], preferred_element_type=jnp.float32) + bd2_ref[...]
    out_ref[...] = x + _ln(y, gd_ref[...], bgd_ref[...])


def _mesh_post(agg0, agg1, mesh_nfeat, Wd1, bd1, Wd2, bd2, gd, bgd):
    row = lambda i: (i, 0)
    full = lambda i: (0, 0)
    wspec = pl.BlockSpec((_D, _D), full)
    vspec = pl.BlockSpec((1, _D), full)
    aspec = pl.BlockSpec((2, _GBLK, _D), lambda i: (0, i, 0))
    return pl.pallas_call(
        _mesh_post_body,
        grid=(_NM // _GBLK,),
        in_specs=[aspec, aspec,
                  pl.BlockSpec((_GBLK, _D), row),
                  wspec, wspec, vspec, wspec, vspec, vspec, vspec],
        out_specs=pl.BlockSpec((_GBLK, _D), row),
        out_shape=jax.ShapeDtypeStruct((_NM, _D), jnp.float32),
        compiler_params=pltpu.CompilerParams(
            dimension_semantics=("parallel",)),
    )(agg0, agg1, mesh_nfeat, Wd1[:_D], Wd1[_D:], bd1.reshape(1, _D), Wd2,
      bd2.reshape(1, _D), gd.reshape(1, _D), bgd.reshape(1, _D))


def kernel(g2m_efeat, grid_nfeat, mesh_nfeat, We, Ws, Wd, be1, We2, be2, ge,
           bge, Ws1, bs1, Ws2, bs2, gs, bgs, Wd1, bd1, Wd2, bd2, gd, bgd,
           src_idx, dst_idx):
    grid_proj = _grid_proj_kernel(grid_nfeat, Ws)
    mesh_proj = _mesh_pre(mesh_nfeat, Wd, be1)
    zeros_nm = jnp.zeros((_NM, _D), jnp.float32)

    gsrc0 = _sc_gather(grid_proj, src_idx[:_EH])
    gdst0 = _sc_gather_spmem(mesh_proj, dst_idx[:_EH])
    grid_out = _grid_branch(grid_nfeat, Ws1, bs1, Ws2, bs2, gs, bgs)
    mlp0 = _edge_mlp(g2m_efeat[:_EH], gsrc0, gdst0, We, We2, be2, ge, bge)
    gsrc1 = _sc_gather(grid_proj, src_idx[_EH:])
    gdst1 = _sc_gather_spmem(mesh_proj, dst_idx[_EH:])
    agg0 = _sc_scatter(mlp0, dst_idx[:_EH], zeros_nm)
    mlp1 = _edge_mlp(g2m_efeat[_EH:], gsrc1, gdst1, We, We2, be2, ge, bge)
    agg1 = _sc_scatter(mlp1, dst_idx[_EH:], zeros_nm)
    mesh_out = _mesh_post(agg0, agg1, mesh_nfeat, Wd1, bd1, Wd2, bd2, gd, bgd)
    return (grid_out, mesh_out)


# speedup vs baseline: 1.0086x; 1.0086x over previous
import functools

import jax
import jax.numpy as jnp
from jax import lax
from jax.experimental import pallas as pl
from jax.experimental.pallas import tpu as pltpu
from jax.experimental.pallas import tpu_sc as plsc

_E = 320000
_NG = 50000
_NM = 10000
_D = 128

_B = 1280
_W = 128
_EH = _E // 2
_NB = _EH // _B

_GBLK = 1000
_C = 128
_KBUF = 3
_NW = 32
_NMT = 624
_NMR = _NM - 16 * _NMT


def _ln(y, g, b):
    m = jnp.mean(y, axis=-1, keepdims=True)
    c = y - m
    v = jnp.mean(c * c, axis=-1, keepdims=True)
    return c * lax.rsqrt(v + 1e-5) * g + b


def _proj_body(x_ref, w_ref, out_ref):
    out_ref[...] = jnp.dot(x_ref[...], w_ref[...],
                           preferred_element_type=jnp.float32)


def _grid_proj_kernel(grid_nfeat, Ws):
    n = grid_nfeat.shape[0]
    return pl.pallas_call(
        _proj_body,
        grid=(n // _GBLK,),
        in_specs=[pl.BlockSpec((_GBLK, _D), lambda i: (i, 0)),
                  pl.BlockSpec((_D, _D), lambda i: (0, 0))],
        out_specs=pl.BlockSpec((_GBLK, _D), lambda i: (i, 0)),
        out_shape=jax.ShapeDtypeStruct((n, _D), jnp.float32),
        compiler_params=pltpu.CompilerParams(
            dimension_semantics=("parallel",)),
    )(grid_nfeat, Ws)


def _mesh_pre_body(x_ref, wd_ref, be1_ref, out_ref):
    out_ref[...] = (jnp.dot(x_ref[...], wd_ref[...],
                            preferred_element_type=jnp.float32) + be1_ref[...])


def _mesh_pre(mesh_nfeat, Wd, be1):
    n = mesh_nfeat.shape[0]
    return pl.pallas_call(
        _mesh_pre_body,
        grid=(n // _GBLK,),
        in_specs=[pl.BlockSpec((_GBLK, _D), lambda i: (i, 0)),
                  pl.BlockSpec((_D, _D), lambda i: (0, 0)),
                  pl.BlockSpec((1, _D), lambda i: (0, 0))],
        out_specs=pl.BlockSpec((_GBLK, _D), lambda i: (i, 0)),
        out_shape=jax.ShapeDtypeStruct((n, _D), jnp.float32),
        compiler_params=pltpu.CompilerParams(
            dimension_semantics=("parallel",)),
    )(mesh_nfeat, Wd, be1.reshape(1, _D))


def _grid_body(x_ref, ws1_ref, bs1_ref, ws2_ref, bs2_ref, gs_ref, bgs_ref,
               out_ref):
    x = x_ref[...]
    hs = jax.nn.silu(
        jnp.dot(x, ws1_ref[...], preferred_element_type=jnp.float32)
        + bs1_ref[...])
    y = jnp.dot(hs, ws2_ref[...], preferred_element_type=jnp.float32) + bs2_ref[...]
    out_ref[...] = x + _ln(y, gs_ref[...], bgs_ref[...])


def _grid_branch(grid_nfeat, Ws1, bs1, Ws2, bs2, gs, bgs):
    n = grid_nfeat.shape[0]
    full = lambda i: (0, 0)
    wspec = pl.BlockSpec((_D, _D), full)
    vspec = pl.BlockSpec((1, _D), full)
    return pl.pallas_call(
        _grid_body,
        grid=(n // _GBLK,),
        in_specs=[pl.BlockSpec((_GBLK, _D), lambda i: (i, 0)), wspec, vspec,
                  wspec, vspec, vspec, vspec],
        out_specs=pl.BlockSpec((_GBLK, _D), lambda i: (i, 0)),
        out_shape=jax.ShapeDtypeStruct((n, _D), jnp.float32),
        compiler_params=pltpu.CompilerParams(
            dimension_semantics=("parallel",)),
    )(grid_nfeat, Ws1, bs1.reshape(1, _D), Ws2, bs2.reshape(1, _D),
      gs.reshape(1, _D), bgs.reshape(1, _D))


def _gather_loop(per_w, n_groups, tail, table, idx_hbm, out_hbm,
                 idx_v, rows_v, tail_idx, tail_rows, sem_g, sem_o, wid):
    base = wid * per_w

    def group(g, carry):
        @pl.when(g > 0)
        def _():
            for b in range(_KBUF):
                off = base + ((g - 1) * _KBUF + b) * _C
                pltpu.make_async_copy(
                    rows_v.at[b], out_hbm.at[pl.ds(off, _C)], sem_o).wait()

        for b in range(_KBUF):
            off = base + (g * _KBUF + b) * _C
            pltpu.sync_copy(idx_hbm.at[pl.ds(off, _C)], idx_v.at[b])
        for b in range(_KBUF):
            pltpu.make_async_copy(
                table.at[idx_v.at[b]], rows_v.at[b], sem_g).start()
        for b in range(_KBUF):
            pltpu.make_async_copy(
                table.at[idx_v.at[b]], rows_v.at[b], sem_g).wait()
        for b in range(_KBUF):
            off = base + (g * _KBUF + b) * _C
            pltpu.make_async_copy(
                rows_v.at[b], out_hbm.at[pl.ds(off, _C)], sem_o).start()
        return carry

    lax.fori_loop(0, n_groups, group, 0)
    for b in range(_KBUF):
        off = base + ((n_groups - 1) * _KBUF + b) * _C
        pltpu.make_async_copy(
            rows_v.at[b], out_hbm.at[pl.ds(off, _C)], sem_o).wait()
    if tail:
        off = base + n_groups * _KBUF * _C
        pltpu.sync_copy(idx_hbm.at[pl.ds(off, tail)], tail_idx)
        pltpu.async_copy(table.at[tail_idx], tail_rows, sem_g).wait()
        pltpu.sync_copy(tail_rows, out_hbm.at[pl.ds(off, tail)])


def _gather_split(n):
    per_w = n // _NW
    n_groups = per_w // (_C * _KBUF)
    tail = per_w - n_groups * _KBUF * _C
    return per_w, n_groups, tail


def _sc_gather_body(per_w, n_groups, tail, table_hbm, idx_hbm, out_hbm,
                    idx_v, rows_v, tail_idx, tail_rows, sem_g, sem_o):
    wid = lax.axis_index("s") * 2 + lax.axis_index("c")
    _gather_loop(per_w, n_groups, tail, table_hbm, idx_hbm, out_hbm,
                 idx_v, rows_v, tail_idx, tail_rows, sem_g, sem_o, wid)


def _sc_gather(table, idx):
    n = idx.shape[0]
    d = table.shape[1]
    per_w, n_groups, tail = _gather_split(n)
    mesh = plsc.VectorSubcoreMesh(core_axis_name="c", subcore_axis_name="s")
    kern = functools.partial(
        pl.kernel,
        mesh=mesh,
        out_type=jax.ShapeDtypeStruct((n, d), table.dtype),
        scratch_types=[
            pltpu.VMEM((_KBUF, _C), jnp.int32),
            pltpu.VMEM((_KBUF, _C, d), table.dtype),
            pltpu.VMEM((max(tail, 8),), jnp.int32),
            pltpu.VMEM((max(tail, 8), d), table.dtype),
            pltpu.SemaphoreType.DMA,
            pltpu.SemaphoreType.DMA,
        ],
    )(functools.partial(_sc_gather_body, per_w, n_groups, tail))
    return kern(table, idx)


def _sc_gather_sp_body(per_w, n_groups, tail, table_hbm, idx_hbm,
                       out_hbm, idx_v, rows_v, tail_idx, tail_rows, table_sh,
                       sem_g, sem_o):
    sid = lax.axis_index("s")
    wid = sid * 2 + lax.axis_index("c")

    pltpu.sync_copy(table_hbm.at[pl.ds(sid * _NMT, _NMT)],
                    table_sh.at[pl.ds(sid * _NMT, _NMT)])

    @pl.when(sid == 15)
    def _():
        pltpu.sync_copy(table_hbm.at[pl.ds(16 * _NMT, _NMR)],
                        table_sh.at[pl.ds(16 * _NMT, _NMR)])

    plsc.subcore_barrier()
    _gather_loop(per_w, n_groups, tail, table_sh, idx_hbm, out_hbm,
                 idx_v, rows_v, tail_idx, tail_rows, sem_g, sem_o, wid)


def _sc_gather_spmem(table, idx):
    n = idx.shape[0]
    d = table.shape[1]
    per_w, n_groups, tail = _gather_split(n)
    mesh = plsc.VectorSubcoreMesh(core_axis_name="c", subcore_axis_name="s")
    kern = functools.partial(
        pl.kernel,
        mesh=mesh,
        out_type=jax.ShapeDtypeStruct((n, d), table.dtype),
        scratch_types=[
            pltpu.VMEM((_KBUF, _C), jnp.int32),
            pltpu.VMEM((_KBUF, _C, d), table.dtype),
            pltpu.VMEM((max(tail, 8),), jnp.int32),
            pltpu.VMEM((max(tail, 8), d), table.dtype),
            pltpu.VMEM_SHARED((table.shape[0], d), table.dtype),
            pltpu.SemaphoreType.DMA,
            pltpu.SemaphoreType.DMA,
        ],
    )(functools.partial(_sc_gather_sp_body, per_w, n_groups, tail))
    return kern(table, idx)


def _sc_scatter_body(per_w, n_groups, tail, mlp_hbm, dst_hbm, zero_hbm,
                     out_hbm, idx_v, rows_v, tail_idx, tail_rows, acc_sh,
                     sem_g):
    cid = lax.axis_index("c")
    sid = lax.axis_index("s")
    wid = cid * 16 + sid
    base = wid * per_w

    pltpu.sync_copy(zero_hbm.at[pl.ds(sid * _NMT, _NMT)],
                    acc_sh.at[pl.ds(sid * _NMT, _NMT)])

    @pl.when(sid == 15)
    def _():
        pltpu.sync_copy(zero_hbm.at[pl.ds(16 * _NMT, _NMR)],
                        acc_sh.at[pl.ds(16 * _NMT, _NMR)])

    plsc.subcore_barrier()

    def group(g, carry):
        for b in range(_KBUF):
            off = base + (g * _KBUF + b) * _C
            pltpu.sync_copy(dst_hbm.at[pl.ds(off, _C)], idx_v.at[b])
        for b in range(_KBUF):
            off = base + (g * _KBUF + b) * _C
            pltpu.make_async_copy(
                mlp_hbm.at[pl.ds(off, _C)], rows_v.at[b], sem_g).start()
        for b in range(_KBUF):
            off = base + (g * _KBUF + b) * _C
            pltpu.make_async_copy(
                mlp_hbm.at[pl.ds(off, _C)], rows_v.at[b], sem_g).wait()
            pltpu.sync_copy(rows_v.at[b], acc_sh.at[idx_v.at[b]], add=True)
        return carry

    lax.fori_loop(0, n_groups, group, 0)
    if tail:
        off = base + n_groups * _KBUF * _C
        pltpu.sync_copy(dst_hbm.at[pl.ds(off, tail)], tail_idx)
        pltpu.sync_copy(mlp_hbm.at[pl.ds(off, tail)], tail_rows)
        pltpu.sync_copy(tail_rows, acc_sh.at[tail_idx], add=True)
    plsc.subcore_barrier()
    pltpu.sync_copy(acc_sh.at[pl.ds(sid * _NMT, _NMT)],
                    out_hbm.at[cid, pl.ds(sid * _NMT, _NMT)])

    @pl.when(sid == 15)
    def _():
        pltpu.sync_copy(acc_sh.at[pl.ds(16 * _NMT, _NMR)],
                        out_hbm.at[cid, pl.ds(16 * _NMT, _NMR)])


def _sc_scatter(mlp, dst_idx_half, zeros_nm):
    n = mlp.shape[0]
    per_w = n // _NW
    n_groups = per_w // (_C * _KBUF)
    tail = per_w - n_groups * _KBUF * _C
    mesh = plsc.VectorSubcoreMesh(core_axis_name="c", subcore_axis_name="s")
    kern = functools.partial(
        pl.kernel,
        mesh=mesh,
        out_type=jax.ShapeDtypeStruct((2, _NM, _D), jnp.float32),
        scratch_types=[
            pltpu.VMEM((_KBUF, _C), jnp.int32),
            pltpu.VMEM((_KBUF, _C, _D), jnp.float32),
            pltpu.VMEM((max(tail, 8),), jnp.int32),
            pltpu.VMEM((max(tail, 8), _D), jnp.float32),
            pltpu.VMEM_SHARED((_NM, _D), jnp.float32),
            pltpu.SemaphoreType.DMA,
        ],
    )(functools.partial(_sc_scatter_body, per_w, n_groups, tail))
    return kern(mlp, dst_idx_half, zeros_nm)


def _edge_body(e_ref, gsrc_ref, gdst_ref, we_ref, ws_ref, we2_ref, be2_ref,
               ge_ref, bge_ref, out_ref):
    we_bf = we_ref[...].astype(jnp.bfloat16)
    ws_bf = ws_ref[...].astype(jnp.bfloat16)
    we2_bf = we2_ref[...].astype(jnp.bfloat16)
    e_bf = e_ref[...].astype(jnp.bfloat16)
    gsrc_bf = gsrc_ref[...].astype(jnp.bfloat16)
    h = jax.nn.silu(
        jnp.dot(e_bf, we_bf, preferred_element_type=jnp.float32)
        + jnp.dot(gsrc_bf, ws_bf, preferred_element_type=jnp.float32)
        + gdst_ref[...])
    y = (jnp.dot(h.astype(jnp.bfloat16), we2_bf,
                 preferred_element_type=jnp.float32) + be2_ref[...])
    out_ref[...] = _ln(y, ge_ref[...], bge_ref[...])


def _edge_mlp(e_half, gsrc, gdst, We, Ws, We2, be2, ge, bge):
    row = lambda i: (i, 0)
    full = lambda i: (0, 0)
    return pl.pallas_call(
        _edge_body,
        grid=(_NB,),
        in_specs=[
            pl.BlockSpec((_B, _D), row),
            pl.BlockSpec((_B, _D), row),
            pl.BlockSpec((_B, _D), row),
            pl.BlockSpec((_D, _D), full),
            pl.BlockSpec((_D, _D), full),
            pl.BlockSpec((_D, _D), full),
            pl.BlockSpec((1, _D), full),
            pl.BlockSpec((1, _D), full),
            pl.BlockSpec((1, _D), full),
        ],
        out_specs=pl.BlockSpec((_B, _D), row),
        out_shape=jax.ShapeDtypeStruct((e_half.shape[0], _D), jnp.float32),
        compiler_params=pltpu.CompilerParams(
            dimension_semantics=("parallel",)),
    )(e_half, gsrc, gdst, We, Ws, We2, be2.reshape(1, _D), ge.reshape(1, _D),
      bge.reshape(1, _D))


def _mesh_post_body(agg0_ref, agg1_ref, x_ref, wd1a_ref, wd1b_ref, bd1_ref,
                    wd2_ref, bd2_ref, gd_ref, bgd_ref, out_ref):
    x = x_ref[...]
    agg = (agg0_ref[0] + agg0_ref[1]) + (agg1_ref[0] + agg1_ref[1])
    hd = jax.nn.silu(
        jnp.dot(agg, wd1a_ref[...], preferred_element_type=jnp.float32)
        + jnp.dot(x, wd1b_ref[...], preferred_element_type=jnp.float32)
        + bd1_ref[...])
    y = jnp.dot(hd, wd2_ref[...], preferred_element_type=jnp.float32) + bd2_ref[...]
    out_ref[...] = x + _ln(y, gd_ref[...], bgd_ref[...])


def _mesh_post(agg0, agg1, mesh_nfeat, Wd1, bd1, Wd2, bd2, gd, bgd):
    row = lambda i: (i, 0)
    full = lambda i: (0, 0)
    wspec = pl.BlockSpec((_D, _D), full)
    vspec = pl.BlockSpec((1, _D), full)
    aspec = pl.BlockSpec((2, _GBLK, _D), lambda i: (0, i, 0))
    return pl.pallas_call(
        _mesh_post_body,
        grid=(_NM // _GBLK,),
        in_specs=[aspec, aspec,
                  pl.BlockSpec((_GBLK, _D), row),
                  wspec, wspec, vspec, wspec, vspec, vspec, vspec],
        out_specs=pl.BlockSpec((_GBLK, _D), row),
        out_shape=jax.ShapeDtypeStruct((_NM, _D), jnp.float32),
        compiler_params=pltpu.CompilerParams(
            dimension_semantics=("parallel",)),
    )(agg0, agg1, mesh_nfeat, Wd1[:_D], Wd1[_D:], bd1.reshape(1, _D), Wd2,
      bd2.reshape(1, _D), gd.reshape(1, _D), bgd.reshape(1, _D))


def kernel(g2m_efeat, grid_nfeat, mesh_nfeat, We, Ws, Wd, be1, We2, be2, ge,
           bge, Ws1, bs1, Ws2, bs2, gs, bgs, Wd1, bd1, Wd2, bd2, gd, bgd,
           src_idx, dst_idx):
    mesh_proj = _mesh_pre(mesh_nfeat, Wd, be1)
    zeros_nm = jnp.zeros((_NM, _D), jnp.float32)

    gsrc0 = _sc_gather(grid_nfeat, src_idx[:_EH])
    gdst0 = _sc_gather_spmem(mesh_proj, dst_idx[:_EH])
    grid_out = _grid_branch(grid_nfeat, Ws1, bs1, Ws2, bs2, gs, bgs)
    mlp0 = _edge_mlp(g2m_efeat[:_EH], gsrc0, gdst0, We, Ws, We2, be2, ge, bge)
    gsrc1 = _sc_gather(grid_nfeat, src_idx[_EH:])
    gdst1 = _sc_gather_spmem(mesh_proj, dst_idx[_EH:])
    agg0 = _sc_scatter(mlp0, dst_idx[:_EH], zeros_nm)
    mlp1 = _edge_mlp(g2m_efeat[_EH:], gsrc1, gdst1, We, Ws, We2, be2, ge, bge)
    agg1 = _sc_scatter(mlp1, dst_idx[_EH:], zeros_nm)
    mesh_out = _mesh_post(agg0, agg1, mesh_nfeat, Wd1, bd1, Wd2, bd2, gd, bgd)
    return (grid_out, mesh_out)


# speedup vs baseline: 1.0426x; 1.0337x over previous
import functools

import jax
import jax.numpy as jnp
from jax import lax
from jax.experimental import pallas as pl
from jax.experimental.pallas import tpu as pltpu
from jax.experimental.pallas import tpu_sc as plsc

_E = 320000
_NG = 50000
_NM = 10000
_D = 128

_B = 1280
_W = 128
_EH = _E // 2
_NB = _EH // _B

_GBLK = 1000
_C = 128
_KBUF = 3
_NW = 32
_NMT = 624
_NMR = _NM - 16 * _NMT


def _ln(y, g, b):
    m = jnp.mean(y, axis=-1, keepdims=True)
    c = y - m
    v = jnp.mean(c * c, axis=-1, keepdims=True)
    return c * lax.rsqrt(v + 1e-5) * g + b


def _proj_body(x_ref, w_ref, out_ref):
    out_ref[...] = jnp.dot(x_ref[...], w_ref[...],
                           preferred_element_type=jnp.float32)


def _grid_proj_kernel(grid_nfeat, Ws):
    n = grid_nfeat.shape[0]
    return pl.pallas_call(
        _proj_body,
        grid=(n // _GBLK,),
        in_specs=[pl.BlockSpec((_GBLK, _D), lambda i: (i, 0)),
                  pl.BlockSpec((_D, _D), lambda i: (0, 0))],
        out_specs=pl.BlockSpec((_GBLK, _D), lambda i: (i, 0)),
        out_shape=jax.ShapeDtypeStruct((n, _D), jnp.float32),
        compiler_params=pltpu.CompilerParams(
            dimension_semantics=("parallel",)),
    )(grid_nfeat, Ws)


def _mesh_pre_body(x_ref, wd_ref, be1_ref, out_ref):
    out_ref[...] = (jnp.dot(x_ref[...], wd_ref[...],
                            preferred_element_type=jnp.float32) + be1_ref[...])


def _mesh_pre(mesh_nfeat, Wd, be1):
    n = mesh_nfeat.shape[0]
    return pl.pallas_call(
        _mesh_pre_body,
        grid=(n // _GBLK,),
        in_specs=[pl.BlockSpec((_GBLK, _D), lambda i: (i, 0)),
                  pl.BlockSpec((_D, _D), lambda i: (0, 0)),
                  pl.BlockSpec((1, _D), lambda i: (0, 0))],
        out_specs=pl.BlockSpec((_GBLK, _D), lambda i: (i, 0)),
        out_shape=jax.ShapeDtypeStruct((n, _D), jnp.float32),
        compiler_params=pltpu.CompilerParams(
            dimension_semantics=("parallel",)),
    )(mesh_nfeat, Wd, be1.reshape(1, _D))


def _grid_body(x_ref, ws1_ref, bs1_ref, ws2_ref, bs2_ref, gs_ref, bgs_ref,
               out_ref):
    x = x_ref[...]
    hs = jax.nn.silu(
        jnp.dot(x, ws1_ref[...], preferred_element_type=jnp.float32)
        + bs1_ref[...])
    y = jnp.dot(hs, ws2_ref[...], preferred_element_type=jnp.float32) + bs2_ref[...]
    out_ref[...] = x + _ln(y, gs_ref[...], bgs_ref[...])


def _grid_branch(grid_nfeat, Ws1, bs1, Ws2, bs2, gs, bgs):
    n = grid_nfeat.shape[0]
    full = lambda i: (0, 0)
    wspec = pl.BlockSpec((_D, _D), full)
    vspec = pl.BlockSpec((1, _D), full)
    return pl.pallas_call(
        _grid_body,
        grid=(n // _GBLK,),
        in_specs=[pl.BlockSpec((_GBLK, _D), lambda i: (i, 0)), wspec, vspec,
                  wspec, vspec, vspec, vspec],
        out_specs=pl.BlockSpec((_GBLK, _D), lambda i: (i, 0)),
        out_shape=jax.ShapeDtypeStruct((n, _D), jnp.float32),
        compiler_params=pltpu.CompilerParams(
            dimension_semantics=("parallel",)),
    )(grid_nfeat, Ws1, bs1.reshape(1, _D), Ws2, bs2.reshape(1, _D),
      gs.reshape(1, _D), bgs.reshape(1, _D))


def _gather_loop(per_w, n_groups, tail, table, idx_hbm, out_hbm,
                 idx_v, rows_v, tail_idx, tail_rows, sem_g, sem_o, wid,
                 prefetch):
    base = wid * per_w

    if prefetch:
        for b in range(_KBUF):
            off = base + b * _C
            pltpu.sync_copy(idx_hbm.at[pl.ds(off, _C)], idx_v.at[b])

    def group(g, carry):
        sl = (g % 2) * _KBUF if prefetch else 0

        @pl.when(g > 0)
        def _():
            for b in range(_KBUF):
                off = base + ((g - 1) * _KBUF + b) * _C
                pltpu.make_async_copy(
                    rows_v.at[b], out_hbm.at[pl.ds(off, _C)], sem_o).wait()

        if not prefetch:
            for b in range(_KBUF):
                off = base + (g * _KBUF + b) * _C
                pltpu.sync_copy(idx_hbm.at[pl.ds(off, _C)], idx_v.at[b])

        for b in range(_KBUF):
            pltpu.make_async_copy(
                table.at[idx_v.at[sl + b]], rows_v.at[b], sem_g).start()

        if prefetch:
            @pl.when(g < n_groups - 1)
            def _():
                nsl = ((g + 1) % 2) * _KBUF
                for b in range(_KBUF):
                    off = base + ((g + 1) * _KBUF + b) * _C
                    pltpu.sync_copy(idx_hbm.at[pl.ds(off, _C)],
                                    idx_v.at[nsl + b])

        for b in range(_KBUF):
            pltpu.make_async_copy(
                table.at[idx_v.at[sl + b]], rows_v.at[b], sem_g).wait()
        for b in range(_KBUF):
            off = base + (g * _KBUF + b) * _C
            pltpu.make_async_copy(
                rows_v.at[b], out_hbm.at[pl.ds(off, _C)], sem_o).start()
        return carry

    lax.fori_loop(0, n_groups, group, 0)
    for b in range(_KBUF):
        off = base + ((n_groups - 1) * _KBUF + b) * _C
        pltpu.make_async_copy(
            rows_v.at[b], out_hbm.at[pl.ds(off, _C)], sem_o).wait()
    if tail:
        off = base + n_groups * _KBUF * _C
        pltpu.sync_copy(idx_hbm.at[pl.ds(off, tail)], tail_idx)
        pltpu.async_copy(table.at[tail_idx], tail_rows, sem_g).wait()
        pltpu.sync_copy(tail_rows, out_hbm.at[pl.ds(off, tail)])


def _gather_split(n):
    per_w = n // _NW
    n_groups = per_w // (_C * _KBUF)
    tail = per_w - n_groups * _KBUF * _C
    return per_w, n_groups, tail


def _sc_gather_body(per_w, n_groups, tail, table_hbm, idx_hbm, out_hbm,
                    idx_v, rows_v, tail_idx, tail_rows, sem_g, sem_o):
    wid = lax.axis_index("s") * 2 + lax.axis_index("c")
    _gather_loop(per_w, n_groups, tail, table_hbm, idx_hbm, out_hbm,
                 idx_v, rows_v, tail_idx, tail_rows, sem_g, sem_o, wid,
                 prefetch=True)


def _sc_gather(table, idx):
    n = idx.shape[0]
    d = table.shape[1]
    per_w, n_groups, tail = _gather_split(n)
    mesh = plsc.VectorSubcoreMesh(core_axis_name="c", subcore_axis_name="s")
    kern = functools.partial(
        pl.kernel,
        mesh=mesh,
        out_type=jax.ShapeDtypeStruct((n, d), table.dtype),
        scratch_types=[
            pltpu.VMEM((2 * _KBUF, _C), jnp.int32),
            pltpu.VMEM((_KBUF, _C, d), table.dtype),
            pltpu.VMEM((max(tail, 8),), jnp.int32),
            pltpu.VMEM((max(tail, 8), d), table.dtype),
            pltpu.SemaphoreType.DMA,
            pltpu.SemaphoreType.DMA,
        ],
    )(functools.partial(_sc_gather_body, per_w, n_groups, tail))
    return kern(table, idx)


def _sc_gather_sp_body(per_w, n_groups, tail, table_hbm, idx_hbm,
                       out_hbm, idx_v, rows_v, tail_idx, tail_rows, table_sh,
                       sem_g, sem_o):
    sid = lax.axis_index("s")
    wid = sid * 2 + lax.axis_index("c")

    pltpu.sync_copy(table_hbm.at[pl.ds(sid * _NMT, _NMT)],
                    table_sh.at[pl.ds(sid * _NMT, _NMT)])

    @pl.when(sid == 15)
    def _():
        pltpu.sync_copy(table_hbm.at[pl.ds(16 * _NMT, _NMR)],
                        table_sh.at[pl.ds(16 * _NMT, _NMR)])

    plsc.subcore_barrier()
    _gather_loop(per_w, n_groups, tail, table_sh, idx_hbm, out_hbm,
                 idx_v, rows_v, tail_idx, tail_rows, sem_g, sem_o, wid,
                 prefetch=False)


def _sc_gather_spmem(table, idx):
    n = idx.shape[0]
    d = table.shape[1]
    per_w, n_groups, tail = _gather_split(n)
    mesh = plsc.VectorSubcoreMesh(core_axis_name="c", subcore_axis_name="s")
    kern = functools.partial(
        pl.kernel,
        mesh=mesh,
        out_type=jax.ShapeDtypeStruct((n, d), table.dtype),
        scratch_types=[
            pltpu.VMEM((_KBUF, _C), jnp.int32),
            pltpu.VMEM((_KBUF, _C, d), table.dtype),
            pltpu.VMEM((max(tail, 8),), jnp.int32),
            pltpu.VMEM((max(tail, 8), d), table.dtype),
            pltpu.VMEM_SHARED((table.shape[0], d), table.dtype),
            pltpu.SemaphoreType.DMA,
            pltpu.SemaphoreType.DMA,
        ],
    )(functools.partial(_sc_gather_sp_body, per_w, n_groups, tail))
    return kern(table, idx)


def _sc_scatter_body(per_w, n_groups, tail, mlp_hbm, dst_hbm, zero_hbm,
                     out_hbm, idx_v, rows_v, tail_idx, tail_rows, acc_sh,
                     sem_g):
    cid = lax.axis_index("c")
    sid = lax.axis_index("s")
    wid = cid * 16 + sid
    base = wid * per_w

    pltpu.sync_copy(zero_hbm.at[pl.ds(sid * _NMT, _NMT)],
                    acc_sh.at[pl.ds(sid * _NMT, _NMT)])

    @pl.when(sid == 15)
    def _():
        pltpu.sync_copy(zero_hbm.at[pl.ds(16 * _NMT, _NMR)],
                        acc_sh.at[pl.ds(16 * _NMT, _NMR)])

    plsc.subcore_barrier()

    def group(g, carry):
        for b in range(_KBUF):
            off = base + (g * _KBUF + b) * _C
            pltpu.make_async_copy(
                mlp_hbm.at[pl.ds(off, _C)], rows_v.at[b], sem_g).start()
        for b in range(_KBUF):
            off = base + (g * _KBUF + b) * _C
            pltpu.sync_copy(dst_hbm.at[pl.ds(off, _C)], idx_v.at[b])
        for b in range(_KBUF):
            off = base + (g * _KBUF + b) * _C
            pltpu.make_async_copy(
                mlp_hbm.at[pl.ds(off, _C)], rows_v.at[b], sem_g).wait()
            pltpu.sync_copy(rows_v.at[b], acc_sh.at[idx_v.at[b]], add=True)
        return carry

    lax.fori_loop(0, n_groups, group, 0)
    if tail:
        off = base + n_groups * _KBUF * _C
        pltpu.sync_copy(dst_hbm.at[pl.ds(off, tail)], tail_idx)
        pltpu.sync_copy(mlp_hbm.at[pl.ds(off, tail)], tail_rows)
        pltpu.sync_copy(tail_rows, acc_sh.at[tail_idx], add=True)
    plsc.subcore_barrier()
    pltpu.sync_copy(acc_sh.at[pl.ds(sid * _NMT, _NMT)],
                    out_hbm.at[cid, pl.ds(sid * _NMT, _NMT)])

    @pl.when(sid == 15)
    def _():
        pltpu.sync_copy(acc_sh.at[pl.ds(16 * _NMT, _NMR)],
                        out_hbm.at[cid, pl.ds(16 * _NMT, _NMR)])


def _sc_scatter(mlp, dst_idx_half, zeros_nm):
    n = mlp.shape[0]
    per_w = n // _NW
    n_groups = per_w // (_C * _KBUF)
    tail = per_w - n_groups * _KBUF * _C
    mesh = plsc.VectorSubcoreMesh(core_axis_name="c", subcore_axis_name="s")
    kern = functools.partial(
        pl.kernel,
        mesh=mesh,
        out_type=jax.ShapeDtypeStruct((2, _NM, _D), jnp.float32),
        scratch_types=[
            pltpu.VMEM((_KBUF, _C), jnp.int32),
            pltpu.VMEM((_KBUF, _C, _D), jnp.float32),
            pltpu.VMEM((max(tail, 8),), jnp.int32),
            pltpu.VMEM((max(tail, 8), _D), jnp.float32),
            pltpu.VMEM_SHARED((_NM, _D), jnp.float32),
            pltpu.SemaphoreType.DMA,
        ],
    )(functools.partial(_sc_scatter_body, per_w, n_groups, tail))
    return kern(mlp, dst_idx_half, zeros_nm)


def _edge_body(e_ref, gsrc_ref, gdst_ref, we_ref, ws_ref, we2_ref, be2_ref,
               ge_ref, bge_ref, out_ref):
    we_bf = we_ref[...].astype(jnp.bfloat16)
    ws_bf = ws_ref[...].astype(jnp.bfloat16)
    we2_bf = we2_ref[...].astype(jnp.bfloat16)
    e_bf = e_ref[...].astype(jnp.bfloat16)
    gsrc_bf = gsrc_ref[...].astype(jnp.bfloat16)
    h = jax.nn.silu(
        jnp.dot(e_bf, we_bf, preferred_element_type=jnp.float32)
        + jnp.dot(gsrc_bf, ws_bf, preferred_element_type=jnp.float32)
        + gdst_ref[...])
    y = (jnp.dot(h.astype(jnp.bfloat16), we2_bf,
                 preferred_element_type=jnp.float32) + be2_ref[...])
    out_ref[...] = _ln(y, ge_ref[...], bge_ref[...])


def _edge_mlp(e_half, gsrc, gdst, We, Ws, We2, be2, ge, bge):
    row = lambda i: (i, 0)
    full = lambda i: (0, 0)
    return pl.pallas_call(
        _edge_body,
        grid=(_NB,),
        in_specs=[
            pl.BlockSpec((_B, _D), row),
            pl.BlockSpec((_B, _D), row),
            pl.BlockSpec((_B, _D), row),
            pl.BlockSpec((_D, _D), full),
            pl.BlockSpec((_D, _D), full),
            pl.BlockSpec((_D, _D), full),
            pl.BlockSpec((1, _D), full),
            pl.BlockSpec((1, _D), full),
            pl.BlockSpec((1, _D), full),
        ],
        out_specs=pl.BlockSpec((_B, _D), row),
        out_shape=jax.ShapeDtypeStruct((e_half.shape[0], _D), jnp.float32),
        compiler_params=pltpu.CompilerParams(
            dimension_semantics=("parallel",)),
    )(e_half, gsrc, gdst, We, Ws, We2, be2.reshape(1, _D), ge.reshape(1, _D),
      bge.reshape(1, _D))


def _mesh_post_body(agg0_ref, agg1_ref, x_ref, wd1a_ref, wd1b_ref, bd1_ref,
                    wd2_ref, bd2_ref, gd_ref, bgd_ref, out_ref):
    x = x_ref[...]
    agg = (agg0_ref[0] + agg0_ref[1]) + (agg1_ref[0] + agg1_ref[1])
    hd = jax.nn.silu(
        jnp.dot(agg, wd1a_ref[...], preferred_element_type=jnp.float32)
        + jnp.dot(x, wd1b_ref[...], preferred_element_type=jnp.float32)
        + bd1_ref[...])
    y = jnp.dot(hd, wd2_ref[...], preferred_element_type=jnp.float32) + bd2_ref[...]
    out_ref[...] = x + _ln(y, gd_ref[...], bgd_ref[...])


def _mesh_post(agg0, agg1, mesh_nfeat, Wd1, bd1, Wd2, bd2, gd, bgd):
    row = lambda i: (i, 0)
    full = lambda i: (0, 0)
    wspec = pl.BlockSpec((_D, _D), full)
    vspec = pl.BlockSpec((1, _D), full)
    aspec = pl.BlockSpec((2, _GBLK, _D), lambda i: (0, i, 0))
    return pl.pallas_call(
        _mesh_post_body,
        grid=(_NM // _GBLK,),
        in_specs=[aspec, aspec,
                  pl.BlockSpec((_GBLK, _D), row),
                  wspec, wspec, vspec, wspec, vspec, vspec, vspec],
        out_specs=pl.BlockSpec((_GBLK, _D), row),
        out_shape=jax.ShapeDtypeStruct((_NM, _D), jnp.float32),
        compiler_params=pltpu.CompilerParams(
            dimension_semantics=("parallel",)),
    )(agg0, agg1, mesh_nfeat, Wd1[:_D], Wd1[_D:], bd1.reshape(1, _D), Wd2,
      bd2.reshape(1, _D), gd.reshape(1, _D), bgd.reshape(1, _D))


def kernel(g2m_efeat, grid_nfeat, mesh_nfeat, We, Ws, Wd, be1, We2, be2, ge,
           bge, Ws1, bs1, Ws2, bs2, gs, bgs, Wd1, bd1, Wd2, bd2, gd, bgd,
           src_idx, dst_idx):
    mesh_proj = _mesh_pre(mesh_nfeat, Wd, be1)
    zeros_nm = jnp.zeros((_NM, _D), jnp.float32)

    gsrc0 = _sc_gather(grid_nfeat, src_idx[:_EH])
    gdst0 = _sc_gather_spmem(mesh_proj, dst_idx[:_EH])
    grid_out = _grid_branch(grid_nfeat, Ws1, bs1, Ws2, bs2, gs, bgs)
    mlp0 = _edge_mlp(g2m_efeat[:_EH], gsrc0, gdst0, We, Ws, We2, be2, ge, bge)
    gsrc1 = _sc_gather(grid_nfeat, src_idx[_EH:])
    gdst1 = _sc_gather_spmem(mesh_proj, dst_idx[_EH:])
    agg0 = _sc_scatter(mlp0, dst_idx[:_EH], zeros_nm)
    mlp1 = _edge_mlp(g2m_efeat[_EH:], gsrc1, gdst1, We, Ws, We2, be2, ge, bge)
    agg1 = _sc_scatter(mlp1, dst_idx[_EH:], zeros_nm)
    mesh_out = _mesh_post(agg0, agg1, mesh_nfeat, Wd1, bd1, Wd2, bd2, gd, bgd)
    return (grid_out, mesh_out)


# speedup vs baseline: 1.0436x; 1.0010x over previous
import functools

import jax
import jax.numpy as jnp
from jax import lax
from jax.experimental import pallas as pl
from jax.experimental.pallas import tpu as pltpu
from jax.experimental.pallas import tpu_sc as plsc

_E = 320000
_NG = 50000
_NM = 10000
_D = 128

_B = 1280
_W = 128
_EH = _E // 2
_NB = _EH // _B

_GBLK = 1000
_C = 128
_KBUF = 3
_NW = 32
_NMT = 624
_NMR = _NM - 16 * _NMT


def _ln(y, g, b):
    m = jnp.mean(y, axis=-1, keepdims=True)
    c = y - m
    v = jnp.mean(c * c, axis=-1, keepdims=True)
    return c * lax.rsqrt(v + 1e-5) * g + b


def _mesh_pre_body(x_ref, wd_ref, be1_ref, out_ref):
    out_ref[...] = (jnp.dot(x_ref[...], wd_ref[...],
                            preferred_element_type=jnp.float32) + be1_ref[...])


def _mesh_pre(mesh_nfeat, Wd, be1):
    n = mesh_nfeat.shape[0]
    return pl.pallas_call(
        _mesh_pre_body,
        grid=(n // _GBLK,),
        in_specs=[pl.BlockSpec((_GBLK, _D), lambda i: (i, 0)),
                  pl.BlockSpec((_D, _D), lambda i: (0, 0)),
                  pl.BlockSpec((1, _D), lambda i: (0, 0))],
        out_specs=pl.BlockSpec((_GBLK, _D), lambda i: (i, 0)),
        out_shape=jax.ShapeDtypeStruct((n, _D), jnp.float32),
        compiler_params=pltpu.CompilerParams(
            dimension_semantics=("parallel",)),
    )(mesh_nfeat, Wd, be1.reshape(1, _D))


def _grid_body(x_ref, ws1_ref, bs1_ref, ws2_ref, bs2_ref, gs_ref, bgs_ref,
               out_ref):
    x = x_ref[...]
    hs = jax.nn.silu(
        jnp.dot(x, ws1_ref[...], preferred_element_type=jnp.float32)
        + bs1_ref[...])
    y = jnp.dot(hs, ws2_ref[...], preferred_element_type=jnp.float32) + bs2_ref[...]
    out_ref[...] = x + _ln(y, gs_ref[...], bgs_ref[...])


def _grid_branch(grid_nfeat, Ws1, bs1, Ws2, bs2, gs, bgs):
    n = grid_nfeat.shape[0]
    full = lambda i: (0, 0)
    wspec = pl.BlockSpec((_D, _D), full)
    vspec = pl.BlockSpec((1, _D), full)
    return pl.pallas_call(
        _grid_body,
        grid=(n // _GBLK,),
        in_specs=[pl.BlockSpec((_GBLK, _D), lambda i: (i, 0)), wspec, vspec,
                  wspec, vspec, vspec, vspec],
        out_specs=pl.BlockSpec((_GBLK, _D), lambda i: (i, 0)),
        out_shape=jax.ShapeDtypeStruct((n, _D), jnp.float32),
        compiler_params=pltpu.CompilerParams(
            dimension_semantics=("parallel",)),
    )(grid_nfeat, Ws1, bs1.reshape(1, _D), Ws2, bs2.reshape(1, _D),
      gs.reshape(1, _D), bgs.reshape(1, _D))


def _gather_loop(per_w, n_groups, tail, table, idx_hbm, out_hbm,
                 idx_v, rows_v, tail_idx, tail_rows, sem_g, sem_o, wid,
                 prefetch):
    base = wid * per_w

    if prefetch:
        for b in range(_KBUF):
            off = base + b * _C
            pltpu.sync_copy(idx_hbm.at[pl.ds(off, _C)], idx_v.at[b])

    def group(g, carry):
        sl = (g % 2) * _KBUF if prefetch else 0

        @pl.when(g > 0)
        def _():
            for b in range(_KBUF):
                off = base + ((g - 1) * _KBUF + b) * _C
                pltpu.make_async_copy(
                    rows_v.at[b], out_hbm.at[pl.ds(off, _C)], sem_o).wait()

        if not prefetch:
            for b in range(_KBUF):
                off = base + (g * _KBUF + b) * _C
                pltpu.sync_copy(idx_hbm.at[pl.ds(off, _C)], idx_v.at[b])

        for b in range(_KBUF):
            pltpu.make_async_copy(
                table.at[idx_v.at[sl + b]], rows_v.at[b], sem_g).start()

        if prefetch:
            @pl.when(g < n_groups - 1)
            def _():
                nsl = ((g + 1) % 2) * _KBUF
                for b in range(_KBUF):
                    off = base + ((g + 1) * _KBUF + b) * _C
                    pltpu.sync_copy(idx_hbm.at[pl.ds(off, _C)],
                                    idx_v.at[nsl + b])

        for b in range(_KBUF):
            pltpu.make_async_copy(
                table.at[idx_v.at[sl + b]], rows_v.at[b], sem_g).wait()
        for b in range(_KBUF):
            off = base + (g * _KBUF + b) * _C
            pltpu.make_async_copy(
                rows_v.at[b], out_hbm.at[pl.ds(off, _C)], sem_o).start()
        return carry

    lax.fori_loop(0, n_groups, group, 0)
    for b in range(_KBUF):
        off = base + ((n_groups - 1) * _KBUF + b) * _C
        pltpu.make_async_copy(
            rows_v.at[b], out_hbm.at[pl.ds(off, _C)], sem_o).wait()
    if tail:
        off = base + n_groups * _KBUF * _C
        pltpu.sync_copy(idx_hbm.at[pl.ds(off, tail)], tail_idx)
        pltpu.async_copy(table.at[tail_idx], tail_rows, sem_g).wait()
        pltpu.sync_copy(tail_rows, out_hbm.at[pl.ds(off, tail)])


def _gather_split(n):
    per_w = n // _NW
    n_groups = per_w // (_C * _KBUF)
    tail = per_w - n_groups * _KBUF * _C
    return per_w, n_groups, tail


def _sc_gather_body(per_w, n_groups, tail, table_hbm, idx_hbm, out_hbm,
                    idx_v, rows_v, tail_idx, tail_rows, sem_g, sem_o):
    wid = lax.axis_index("s") * 2 + lax.axis_index("c")
    _gather_loop(per_w, n_groups, tail, table_hbm, idx_hbm, out_hbm,
                 idx_v, rows_v, tail_idx, tail_rows, sem_g, sem_o, wid,
                 prefetch=True)


def _sc_gather(table, idx):
    n = idx.shape[0]
    d = table.shape[1]
    per_w, n_groups, tail = _gather_split(n)
    mesh = plsc.VectorSubcoreMesh(core_axis_name="c", subcore_axis_name="s")
    kern = functools.partial(
        pl.kernel,
        mesh=mesh,
        out_type=jax.ShapeDtypeStruct((n, d), table.dtype),
        scratch_types=[
            pltpu.VMEM((2 * _KBUF, _C), jnp.int32),
            pltpu.VMEM((_KBUF, _C, d), table.dtype),
            pltpu.VMEM((max(tail, 8),), jnp.int32),
            pltpu.VMEM((max(tail, 8), d), table.dtype),
            pltpu.SemaphoreType.DMA,
            pltpu.SemaphoreType.DMA,
        ],
    )(functools.partial(_sc_gather_body, per_w, n_groups, tail))
    return kern(table, idx)


def _sc_gather_sp_body(per_w, n_groups, tail, table_hbm, idx_hbm,
                       out_hbm, idx_v, rows_v, tail_idx, tail_rows, table_sh,
                       sem_g, sem_o):
    sid = lax.axis_index("s")
    wid = sid * 2 + lax.axis_index("c")

    pltpu.sync_copy(table_hbm.at[pl.ds(sid * _NMT, _NMT)],
                    table_sh.at[pl.ds(sid * _NMT, _NMT)])

    @pl.when(sid == 15)
    def _():
        pltpu.sync_copy(table_hbm.at[pl.ds(16 * _NMT, _NMR)],
                        table_sh.at[pl.ds(16 * _NMT, _NMR)])

    plsc.subcore_barrier()
    _gather_loop(per_w, n_groups, tail, table_sh, idx_hbm, out_hbm,
                 idx_v, rows_v, tail_idx, tail_rows, sem_g, sem_o, wid,
                 prefetch=False)


def _sc_gather_spmem(table, idx):
    n = idx.shape[0]
    d = table.shape[1]
    per_w, n_groups, tail = _gather_split(n)
    mesh = plsc.VectorSubcoreMesh(core_axis_name="c", subcore_axis_name="s")
    kern = functools.partial(
        pl.kernel,
        mesh=mesh,
        out_type=jax.ShapeDtypeStruct((n, d), table.dtype),
        scratch_types=[
            pltpu.VMEM((_KBUF, _C), jnp.int32),
            pltpu.VMEM((_KBUF, _C, d), table.dtype),
            pltpu.VMEM((max(tail, 8),), jnp.int32),
            pltpu.VMEM((max(tail, 8), d), table.dtype),
            pltpu.VMEM_SHARED((table.shape[0], d), table.dtype),
            pltpu.SemaphoreType.DMA,
            pltpu.SemaphoreType.DMA,
        ],
    )(functools.partial(_sc_gather_sp_body, per_w, n_groups, tail))
    return kern(table, idx)


def _sc_scatter_body(per_w, n_groups, tail, mlp_hbm, dst_hbm, zero_hbm,
                     out_hbm, idx_v, rows_v, tail_idx, tail_rows, acc_sh,
                     sem_g):
    cid = lax.axis_index("c")
    sid = lax.axis_index("s")
    wid = cid * 16 + sid
    base = wid * per_w

    pltpu.sync_copy(zero_hbm.at[pl.ds(sid * _NMT, _NMT)],
                    acc_sh.at[pl.ds(sid * _NMT, _NMT)])

    @pl.when(sid == 15)
    def _():
        pltpu.sync_copy(zero_hbm.at[pl.ds(16 * _NMT, _NMR)],
                        acc_sh.at[pl.ds(16 * _NMT, _NMR)])

    plsc.subcore_barrier()

    def group(g, carry):
        for b in range(_KBUF):
            off = base + (g * _KBUF + b) * _C
            pltpu.make_async_copy(
                mlp_hbm.at[pl.ds(off, _C)], rows_v.at[b], sem_g).start()
        for b in range(_KBUF):
            off = base + (g * _KBUF + b) * _C
            pltpu.sync_copy(dst_hbm.at[pl.ds(off, _C)], idx_v.at[b])
        for b in range(_KBUF):
            off = base + (g * _KBUF + b) * _C
            pltpu.make_async_copy(
                mlp_hbm.at[pl.ds(off, _C)], rows_v.at[b], sem_g).wait()
            pltpu.sync_copy(rows_v.at[b], acc_sh.at[idx_v.at[b]], add=True)
        return carry

    lax.fori_loop(0, n_groups, group, 0)
    if tail:
        off = base + n_groups * _KBUF * _C
        pltpu.sync_copy(dst_hbm.at[pl.ds(off, tail)], tail_idx)
        pltpu.sync_copy(mlp_hbm.at[pl.ds(off, tail)], tail_rows)
        pltpu.sync_copy(tail_rows, acc_sh.at[tail_idx], add=True)
    plsc.subcore_barrier()
    pltpu.sync_copy(acc_sh.at[pl.ds(sid * _NMT, _NMT)],
                    out_hbm.at[cid, pl.ds(sid * _NMT, _NMT)])

    @pl.when(sid == 15)
    def _():
        pltpu.sync_copy(acc_sh.at[pl.ds(16 * _NMT, _NMR)],
                        out_hbm.at[cid, pl.ds(16 * _NMT, _NMR)])


def _sc_scatter(mlp, dst_idx_half, zeros_nm):
    n = mlp.shape[0]
    per_w = n // _NW
    n_groups = per_w // (_C * _KBUF)
    tail = per_w - n_groups * _KBUF * _C
    mesh = plsc.VectorSubcoreMesh(core_axis_name="c", subcore_axis_name="s")
    kern = functools.partial(
        pl.kernel,
        mesh=mesh,
        out_type=jax.ShapeDtypeStruct((2, _NM, _D), jnp.float32),
        scratch_types=[
            pltpu.VMEM((_KBUF, _C), jnp.int32),
            pltpu.VMEM((_KBUF, _C, _D), jnp.float32),
            pltpu.VMEM((max(tail, 8),), jnp.int32),
            pltpu.VMEM((max(tail, 8), _D), jnp.float32),
            pltpu.VMEM_SHARED((_NM, _D), jnp.float32),
            pltpu.SemaphoreType.DMA,
        ],
    )(functools.partial(_sc_scatter_body, per_w, n_groups, tail))
    return kern(mlp, dst_idx_half, zeros_nm)


def _edge_body(e_ref, gsrc_ref, gdst_ref, we_ref, ws_ref, we2_ref, be2_ref,
               ge_ref, bge_ref, out_ref):
    we_bf = we_ref[...].astype(jnp.bfloat16)
    ws_bf = ws_ref[...].astype(jnp.bfloat16)
    we2_bf = we2_ref[...].astype(jnp.bfloat16)
    e_bf = e_ref[...].astype(jnp.bfloat16)
    gsrc_bf = gsrc_ref[...].astype(jnp.bfloat16)
    h = jax.nn.silu(
        jnp.dot(e_bf, we_bf, preferred_element_type=jnp.float32)
        + jnp.dot(gsrc_bf, ws_bf, preferred_element_type=jnp.float32)
        + gdst_ref[...])
    y = (jnp.dot(h.astype(jnp.bfloat16), we2_bf,
                 preferred_element_type=jnp.float32) + be2_ref[...])
    out_ref[...] = _ln(y, ge_ref[...], bge_ref[...])


def _edge_mlp(e_half, gsrc, gdst, We, Ws, We2, be2, ge, bge):
    row = lambda i: (i, 0)
    full = lambda i: (0, 0)
    return pl.pallas_call(
        _edge_body,
        grid=(_NB,),
        in_specs=[
            pl.BlockSpec((_B, _D), row),
            pl.BlockSpec((_B, _D), row),
            pl.BlockSpec((_B, _D), row),
            pl.BlockSpec((_D, _D), full),
            pl.BlockSpec((_D, _D), full),
            pl.BlockSpec((_D, _D), full),
            pl.BlockSpec((1, _D), full),
            pl.BlockSpec((1, _D), full),
            pl.BlockSpec((1, _D), full),
        ],
        out_specs=pl.BlockSpec((_B, _D), row),
        out_shape=jax.ShapeDtypeStruct((e_half.shape[0], _D), jnp.float32),
        compiler_params=pltpu.CompilerParams(
            dimension_semantics=("parallel",)),
    )(e_half, gsrc, gdst, We, Ws, We2, be2.reshape(1, _D), ge.reshape(1, _D),
      bge.reshape(1, _D))


def _mesh_post_body(agg0_ref, agg1_ref, x_ref, wd1a_ref, wd1b_ref, bd1_ref,
                    wd2_ref, bd2_ref, gd_ref, bgd_ref, out_ref):
    x = x_ref[...]
    agg = (agg0_ref[0] + agg0_ref[1]) + (agg1_ref[0] + agg1_ref[1])
    hd = jax.nn.silu(
        jnp.dot(agg, wd1a_ref[...], preferred_element_type=jnp.float32)
        + jnp.dot(x, wd1b_ref[...], preferred_element_type=jnp.float32)
        + bd1_ref[...])
    y = jnp.dot(hd, wd2_ref[...], preferred_element_type=jnp.float32) + bd2_ref[...]
    out_ref[...] = x + _ln(y, gd_ref[...], bgd_ref[...])


def _mesh_post(agg0, agg1, mesh_nfeat, Wd1, bd1, Wd2, bd2, gd, bgd):
    row = lambda i: (i, 0)
    full = lambda i: (0, 0)
    wspec = pl.BlockSpec((_D, _D), full)
    vspec = pl.BlockSpec((1, _D), full)
    aspec = pl.BlockSpec((2, _GBLK, _D), lambda i: (0, i, 0))
    return pl.pallas_call(
        _mesh_post_body,
        grid=(_NM // _GBLK,),
        in_specs=[aspec, aspec,
                  pl.BlockSpec((_GBLK, _D), row),
                  wspec, wspec, vspec, wspec, vspec, vspec, vspec],
        out_specs=pl.BlockSpec((_GBLK, _D), row),
        out_shape=jax.ShapeDtypeStruct((_NM, _D), jnp.float32),
        compiler_params=pltpu.CompilerParams(
            dimension_semantics=("parallel",)),
    )(agg0, agg1, mesh_nfeat, Wd1[:_D], Wd1[_D:], bd1.reshape(1, _D), Wd2,
      bd2.reshape(1, _D), gd.reshape(1, _D), bgd.reshape(1, _D))


def kernel(g2m_efeat, grid_nfeat, mesh_nfeat, We, Ws, Wd, be1, We2, be2, ge,
           bge, Ws1, bs1, Ws2, bs2, gs, bgs, Wd1, bd1, Wd2, bd2, gd, bgd,
           src_idx, dst_idx):
    mesh_proj = _mesh_pre(mesh_nfeat, Wd, be1)
    zeros_nm = jnp.zeros((_NM, _D), jnp.float32)

    gsrc0 = _sc_gather(grid_nfeat, src_idx[:_EH])
    gdst0 = _sc_gather_spmem(mesh_proj, dst_idx[:_EH])
    grid_out = _grid_branch(grid_nfeat, Ws1, bs1, Ws2, bs2, gs, bgs)
    mlp0 = _edge_mlp(g2m_efeat[:_EH], gsrc0, gdst0, We, Ws, We2, be2, ge, bge)
    gsrc1 = _sc_gather(grid_nfeat, src_idx[_EH:])
    gdst1 = _sc_gather_spmem(mesh_proj, dst_idx[_EH:])
    agg0 = _sc_scatter(mlp0, dst_idx[:_EH], zeros_nm)
    mlp1 = _edge_mlp(g2m_efeat[_EH:], gsrc1, gdst1, We, Ws, We2, be2, ge, bge)
    agg1 = _sc_scatter(mlp1, dst_idx[_EH:], zeros_nm)
    mesh_out = _mesh_post(agg0, agg1, mesh_nfeat, Wd1, bd1, Wd2, bd2, gd, bgd)
    return (grid_out, mesh_out)
